# baseline (speedup 1.0000x reference)
import jax
import jax.numpy as jnp
from jax import lax
from jax.experimental import pallas as pl
from jax.experimental.pallas import tpu as pltpu


def _mbl_kernel(ca_ref, cb_ref, la_ref, lb_ref, aa_ref, ab_ref,
                out_ref, m_s, np_s, acc_s):
    i = pl.program_id(0)
    h = pl.num_programs(0)
    num = 2 * h
    nc, d = ca_ref.shape[1], ca_ref.shape[2]
    seg_n = (aa_ref.shape[2] - d) // 2

    def row(conf_ref, aux_ref, loc_ref, r):
        x = conf_ref[0]
        aux = aux_ref[0]
        t = aux[:, :d].astype(jnp.int32)
        s = jnp.sum(jnp.exp(x), axis=0, keepdims=True)
        lse = jnp.log(s)
        cls = lax.broadcasted_iota(jnp.int32, (nc, d), 0)
        g = jnp.sum(jnp.where(cls == t, x, 0.0), axis=0, keepdims=True)
        ce = lse - g
        pos = t > 0
        posf = pos.astype(jnp.float32)
        m = jnp.maximum(jnp.where(pos, 0.0, ce), 0.0)
        m_s[pl.ds(r, 1)] = lax.bitcast_convert_type(m, jnp.int32)[None]
        npos = jnp.sum(posf)
        np_s[pl.ds(r, 1)] = jnp.full((1, 1, 128), npos, jnp.float32)
        lc = loc_ref[0]
        dd = lc[0:4] - lc[4:8]
        ad = jnp.abs(dd)
        sl1 = jnp.where(ad < 1.0, 0.5 * dd * dd, ad - 0.5)
        sl1_sum = jnp.sum(sl1 * posf)
        posce = jnp.sum(ce * posf)
        sdv = aux[:, d:d + seg_n]
        sgv = aux[:, d + seg_n:]
        inter = jnp.sum(sdv * sgv)
        union = jnp.sum(sdv + sgv)
        return sl1_sum, posce, inter, union

    ra = row(ca_ref, aa_ref, la_ref, i)
    rb = row(cb_ref, ab_ref, lb_ref, i + h)

    @pl.when(i == 0)
    def _init():
        acc_s[0] = 0.0
        acc_s[1] = 0.0
        acc_s[2] = 0.0
        acc_s[3] = 0.0

    for j in range(4):
        acc_s[j] = acc_s[j] + ra[j] + rb[j]

    @pl.when(i == h - 1)
    def _final():
        npos_v = np_s[:, :, 0:1]
        k = jnp.minimum(3 * npos_v.astype(jnp.int32), d - 1)

        def body(_, carry):
            lo, hi = carry
            mid = lo + (hi - lo) // 2
            cnt = jnp.sum((m_s[...] > mid).astype(jnp.int32), axis=2,
                          keepdims=True)
            shrink = cnt < k
            return (jnp.where(shrink, lo, mid + 1),
                    jnp.where(shrink, mid, hi))

        lo0 = jnp.zeros((num, 1, 1), jnp.int32)
        hi0 = jnp.full((num, 1, 1), jnp.int32(0x7F800000))
        lo, _ = lax.fori_loop(0, 31, body, (lo0, hi0))
        bits = m_s[...]
        gt = bits > lo
        cnt_gt = jnp.sum(gt.astype(jnp.int32), axis=2, keepdims=True)
        sum_gt = jnp.sum(
            jnp.where(gt, lax.bitcast_convert_type(bits, jnp.float32), 0.0),
            axis=2, keepdims=True)
        tf = jnp.where(k > 0, lax.bitcast_convert_type(lo, jnp.float32), 0.0)
        topk = sum_gt + (k - cnt_gt).astype(jnp.float32) * tf
        topk_total = jnp.sum(topk)
        n_tot = jnp.sum(npos_v)
        out_ref[0] = acc_s[0] / n_tot
        out_ref[1] = (acc_s[1] + topk_total) / n_tot
        out_ref[2] = 1.0 - 2.0 * acc_s[2] / (acc_s[3] + 1e-5)


def kernel(loc_data, conf_data, priors, seg_data, loc_t, conf_t, segs):
    num, p, a, nc = conf_data.shape
    d = p * a
    seg_n = segs.shape[1]
    h = num // 2
    sdr = seg_data.reshape(num, 1, seg_n)
    sgr = segs.reshape(num, 1, seg_n)

    def stage(lo, hi):
        hh = hi - lo
        conf_tr = jnp.swapaxes(conf_data[lo:hi].reshape(hh, d, nc), 1, 2)
        loc_cat = jnp.concatenate(
            [jnp.swapaxes(loc_data[lo:hi].reshape(hh, d, 4), 1, 2),
             jnp.swapaxes(loc_t[lo:hi].reshape(hh, d, 4), 1, 2)], axis=1)
        aux = jnp.concatenate(
            [conf_t[lo:hi].reshape(hh, 1, d).astype(jnp.float32),
             sdr[lo:hi], sgr[lo:hi]], axis=2)
        return conf_tr, loc_cat, aux

    ca, la, aa = stage(0, h)
    cb, lb, ab = stage(h, num)

    out = pl.pallas_call(
        _mbl_kernel,
        grid=(h,),
        in_specs=[
            pl.BlockSpec((1, nc, d), lambda i: (i, 0, 0)),
            pl.BlockSpec((1, nc, d), lambda i: (i, 0, 0)),
            pl.BlockSpec((1, 8, d), lambda i: (i, 0, 0)),
            pl.BlockSpec((1, 8, d), lambda i: (i, 0, 0)),
            pl.BlockSpec((1, 1, d + 2 * seg_n), lambda i: (i, 0, 0)),
            pl.BlockSpec((1, 1, d + 2 * seg_n), lambda i: (i, 0, 0)),
        ],
        out_specs=pl.BlockSpec(memory_space=pltpu.SMEM),
        out_shape=jax.ShapeDtypeStruct((4,), jnp.float32),
        scratch_shapes=[
            pltpu.VMEM((num, 1, d), jnp.int32),
            pltpu.VMEM((num, 1, 128), jnp.float32),
            pltpu.SMEM((4,), jnp.float32),
        ],
    )(ca, cb, la, lb, aa, ab)
    return (out[0], out[1], out[2])


# speedup vs baseline: 2.3628x; 2.3628x over previous
import jax
import jax.numpy as jnp
from jax import lax
from jax.experimental import pallas as pl
from jax.experimental.pallas import tpu as pltpu


def _mbl_kernel(conf_ref, ct_ref, locd_ref, loct_ref, sd_ref, sg_ref,
                out_ref, m_s, np_s, acc_s):
    i = pl.program_id(0)
    num = pl.num_programs(0)
    x = conf_ref[0]
    t = ct_ref[0]
    nc, d = x.shape

    s = jnp.sum(jnp.exp(x), axis=0, keepdims=True)
    lse = jnp.log(s)
    cls = lax.broadcasted_iota(jnp.int32, (nc, d), 0)
    g = jnp.sum(jnp.where(cls == t, x, 0.0), axis=0, keepdims=True)
    ce = lse - g
    pos = t > 0
    posf = pos.astype(jnp.float32)
    m = jnp.maximum(jnp.where(pos, 0.0, ce), 0.0)
    m_s[pl.ds(i, 1)] = lax.bitcast_convert_type(m, jnp.int32)[None]

    npos = jnp.sum(posf)
    np_s[pl.ds(i, 1)] = jnp.full((1, 1, 128), npos, jnp.float32)

    dd = locd_ref[0] - loct_ref[0]
    ad = jnp.abs(dd)
    sl1 = jnp.where(ad < 1.0, 0.5 * dd * dd, ad - 0.5)
    sl1_sum = jnp.sum(sl1 * posf)
    posce = jnp.sum(ce * posf)

    @pl.when(i == 0)
    def _init():
        sdv = sd_ref[...]
        sgv = sg_ref[...]
        acc_s[0] = 0.0
        acc_s[1] = 0.0
        acc_s[2] = jnp.sum(sdv * sgv)
        acc_s[3] = jnp.sum(sdv) + jnp.sum(sgv)

    acc_s[0] = acc_s[0] + sl1_sum
    acc_s[1] = acc_s[1] + posce

    @pl.when(i == num - 1)
    def _final():
        npos_v = np_s[:, :, 0:1]
        k = jnp.minimum(3 * npos_v.astype(jnp.int32), d - 1)

        def body(_, carry):
            lo, hi = carry
            mid = lo + (hi - lo) // 2
            cnt = jnp.sum((m_s[...] > mid).astype(jnp.int32), axis=2,
                          keepdims=True)
            shrink = cnt < k
            return (jnp.where(shrink, lo, mid + 1),
                    jnp.where(shrink, mid, hi))

        lo0 = jnp.zeros((num, 1, 1), jnp.int32)
        hi0 = jnp.full((num, 1, 1), jnp.int32(0x7F800000))
        lo, _ = lax.fori_loop(0, 31, body, (lo0, hi0))
        bits = m_s[...]
        gt = bits > lo
        cnt_gt = jnp.sum(gt.astype(jnp.int32), axis=2, keepdims=True)
        sum_gt = jnp.sum(
            jnp.where(gt, lax.bitcast_convert_type(bits, jnp.float32), 0.0),
            axis=2, keepdims=True)
        tf = jnp.where(k > 0, lax.bitcast_convert_type(lo, jnp.float32), 0.0)
        topk = sum_gt + (k - cnt_gt).astype(jnp.float32) * tf
        topk_total = jnp.sum(topk)
        n_tot = jnp.sum(npos_v)
        out_ref[0] = acc_s[0] / n_tot
        out_ref[1] = (acc_s[1] + topk_total) / n_tot
        out_ref[2] = 1.0 - 2.0 * acc_s[2] / (acc_s[3] + 1e-5)


def kernel(loc_data, conf_data, priors, seg_data, loc_t, conf_t, segs):
    num, p, a, nc = conf_data.shape
    d = p * a
    seg_n = segs.shape[1]
    conf_tr = jnp.swapaxes(conf_data.reshape(num, d, nc), 1, 2)
    locd_tr = jnp.swapaxes(loc_data.reshape(num, d, 4), 1, 2)
    loct_tr = jnp.swapaxes(loc_t.reshape(num, d, 4), 1, 2)
    ct = conf_t.reshape(num, 1, d).astype(jnp.int32)
    sd = seg_data.reshape(num * seg_n // 128, 128)
    sg = segs.reshape(num * seg_n // 128, 128)

    out = pl.pallas_call(
        _mbl_kernel,
        grid=(num,),
        in_specs=[
            pl.BlockSpec((1, nc, d), lambda i: (i, 0, 0)),
            pl.BlockSpec((1, 1, d), lambda i: (i, 0, 0)),
            pl.BlockSpec((1, 4, d), lambda i: (i, 0, 0)),
            pl.BlockSpec((1, 4, d), lambda i: (i, 0, 0)),
            pl.BlockSpec((num * seg_n // 128, 128), lambda i: (0, 0)),
            pl.BlockSpec((num * seg_n // 128, 128), lambda i: (0, 0)),
        ],
        out_specs=pl.BlockSpec(memory_space=pltpu.SMEM),
        out_shape=jax.ShapeDtypeStruct((4,), jnp.float32),
        scratch_shapes=[
            pltpu.VMEM((num, 1, d), jnp.int32),
            pltpu.VMEM((num, 1, 128), jnp.float32),
            pltpu.SMEM((4,), jnp.float32),
        ],
    )(conf_tr, ct, locd_tr, loct_tr, sd, sg)
    return (out[0], out[1], out[2])


# speedup vs baseline: 2.4280x; 1.0276x over previous
import jax
import jax.numpy as jnp
from jax import lax
from jax.experimental import pallas as pl
from jax.experimental.pallas import tpu as pltpu


def _aux_kernel(ct_ref, locd_ref, loct_ref, sd_ref, sg_ref,
                npv_ref, part_ref, acc_s):
    i = pl.program_id(0)
    num = pl.num_programs(0)
    t = ct_ref[0]
    posf = (t > 0).astype(jnp.float32)
    npv_ref[...] = jnp.full(npv_ref.shape, jnp.sum(posf), jnp.float32)
    dd = locd_ref[0] - loct_ref[0]
    ad = jnp.abs(dd)
    sl1 = jnp.where(ad < 1.0, 0.5 * dd * dd, ad - 0.5)
    sl1_sum = jnp.sum(sl1 * posf)

    @pl.when(i == 0)
    def _init():
        sdv = sd_ref[...]
        sgv = sg_ref[...]
        acc_s[0] = 0.0
        acc_s[1] = jnp.sum(sdv * sgv)
        acc_s[2] = jnp.sum(sdv) + jnp.sum(sgv)

    acc_s[0] = acc_s[0] + sl1_sum

    @pl.when(i == num - 1)
    def _emit():
        part_ref[0] = acc_s[0]
        part_ref[1] = acc_s[1]
        part_ref[2] = acc_s[2]
        part_ref[3] = 0.0


def _conf_kernel(conf_ref, ct_ref, npv_ref, part_ref, out_ref, m_s, acc_s):
    i = pl.program_id(0)
    num = pl.num_programs(0)
    x = conf_ref[0]
    t = ct_ref[0]
    nc, d = x.shape

    s = jnp.sum(jnp.exp(x), axis=0, keepdims=True)
    lse = jnp.log(s)
    cls = lax.broadcasted_iota(jnp.int32, (nc, d), 0)
    g = jnp.sum(jnp.where(cls == t, x, 0.0), axis=0, keepdims=True)
    ce = lse - g
    pos = t > 0
    m = jnp.maximum(jnp.where(pos, 0.0, ce), 0.0)
    m_s[pl.ds(i, 1)] = lax.bitcast_convert_type(m, jnp.int32)[None]
    posce = jnp.sum(jnp.where(pos, ce, 0.0))

    @pl.when(i == 0)
    def _init():
        acc_s[0] = 0.0

    acc_s[0] = acc_s[0] + posce

    @pl.when(i == num - 1)
    def _final():
        npos_v = npv_ref[:, :, 0:1]
        k = jnp.minimum(3 * npos_v.astype(jnp.int32), d - 1)

        def body(_, carry):
            lo, hi = carry
            mid = lo + (hi - lo) // 2
            cnt = jnp.sum((m_s[...] > mid).astype(jnp.int32), axis=2,
                          keepdims=True)
            shrink = cnt < k
            return (jnp.where(shrink, lo, mid + 1),
                    jnp.where(shrink, mid, hi))

        lo0 = jnp.zeros((num, 1, 1), jnp.int32)
        hi0 = jnp.full((num, 1, 1), jnp.int32(0x7F800000))
        lo, _ = lax.fori_loop(0, 31, body, (lo0, hi0))
        bits = m_s[...]
        gt = bits > lo
        cnt_gt = jnp.sum(gt.astype(jnp.int32), axis=2, keepdims=True)
        sum_gt = jnp.sum(
            jnp.where(gt, lax.bitcast_convert_type(bits, jnp.float32), 0.0),
            axis=2, keepdims=True)
        tf = jnp.where(k > 0, lax.bitcast_convert_type(lo, jnp.float32), 0.0)
        topk = sum_gt + (k - cnt_gt).astype(jnp.float32) * tf
        topk_total = jnp.sum(topk)
        n_tot = jnp.sum(npos_v)
        out_ref[0] = part_ref[0] / n_tot
        out_ref[1] = (acc_s[0] + topk_total) / n_tot
        out_ref[2] = 1.0 - 2.0 * part_ref[1] / (part_ref[2] + 1e-5)


def kernel(loc_data, conf_data, priors, seg_data, loc_t, conf_t, segs):
    num, p, a, nc = conf_data.shape
    d = p * a
    seg_n = segs.shape[1]
    conf_tr = jnp.swapaxes(conf_data.reshape(num, d, nc), 1, 2)
    locd_tr = jnp.swapaxes(loc_data.reshape(num, d, 4), 1, 2)
    loct_tr = jnp.swapaxes(loc_t.reshape(num, d, 4), 1, 2)
    ct = conf_t.reshape(num, 1, d).astype(jnp.int32)
    sd = seg_data.reshape(num * seg_n // 128, 128)
    sg = segs.reshape(num * seg_n // 128, 128)

    npv, part = pl.pallas_call(
        _aux_kernel,
        grid=(num,),
        in_specs=[
            pl.BlockSpec((1, 1, d), lambda i: (i, 0, 0)),
            pl.BlockSpec((1, 4, d), lambda i: (i, 0, 0)),
            pl.BlockSpec((1, 4, d), lambda i: (i, 0, 0)),
            pl.BlockSpec((num * seg_n // 128, 128), lambda i: (0, 0)),
            pl.BlockSpec((num * seg_n // 128, 128), lambda i: (0, 0)),
        ],
        out_specs=[
            pl.BlockSpec((1, 1, 128), lambda i: (i, 0, 0)),
            pl.BlockSpec(memory_space=pltpu.SMEM),
        ],
        out_shape=[
            jax.ShapeDtypeStruct((num, 1, 128), jnp.float32),
            jax.ShapeDtypeStruct((4,), jnp.float32),
        ],
        scratch_shapes=[pltpu.SMEM((4,), jnp.float32)],
    )(ct, locd_tr, loct_tr, sd, sg)

    out = pl.pallas_call(
        _conf_kernel,
        grid=(num,),
        in_specs=[
            pl.BlockSpec((1, nc, d), lambda i: (i, 0, 0)),
            pl.BlockSpec((1, 1, d), lambda i: (i, 0, 0)),
            pl.BlockSpec((num, 1, 128), lambda i: (0, 0, 0)),
            pl.BlockSpec(memory_space=pltpu.SMEM),
        ],
        out_specs=pl.BlockSpec(memory_space=pltpu.SMEM),
        out_shape=jax.ShapeDtypeStruct((4,), jnp.float32),
        scratch_shapes=[
            pltpu.VMEM((num, 1, d), jnp.int32),
            pltpu.SMEM((4,), jnp.float32),
        ],
    )(conf_tr, ct, npv, part)
    return (out[0], out[1], out[2])
